# baseline (speedup 1.0000x reference)
import functools

import jax
import jax.numpy as jnp
from jax import lax
from jax.experimental import pallas as pl
from jax.experimental.pallas import tpu as pltpu
from jax.experimental.pallas import tpu_sc as plsc

HID = 64
PADH = 128
NC = 2
NS = 16
NW = NC * NS
CHUNK = 200
NBUF = 4
TCH = 128


def _transpose_body(tableT_hbm, wide_hbm, src_v, dst_v, ssem, wsem):
    wid = lax.axis_index("s") * NC + lax.axis_index("c")
    vpad = wide_hbm.shape[0]
    nch = vpad // TCH

    def stage_desc(c, b):
        return pltpu.make_async_copy(
            tableT_hbm.at[:, pl.ds(c * TCH, TCH)], src_v.at[b], ssem.at[b]
        )

    def write_desc(c, b):
        return pltpu.make_async_copy(
            dst_v.at[b], wide_hbm.at[pl.ds(c * TCH, TCH)], wsem.at[b]
        )

    iotas = [lax.iota(jnp.int32, 16) + 16 * j for j in range(4)]

    def transpose_chunk(b):
        def rows(i, carry):
            for dr in range(8):
                r = i * 8 + dr
                rv = jnp.full((16,), 0, jnp.int32) + r
                for j in range(4):
                    vals = plsc.load_gather(src_v.at[b], [iotas[j], rv])
                    dst_v[b, r, pl.ds(16 * j, 16)] = vals
            return carry

        lax.fori_loop(0, TCH // 8, rows, 0)

    stage_desc(wid, 0).start()

    def pair(k2, carry):
        for b in range(2):
            k = 2 * k2 + b
            c = wid + NW * k

            @pl.when(c < nch)
            def _():
                cn = c + NW

                @pl.when(cn < nch)
                def _():
                    stage_desc(cn, 1 - b).start()

                stage_desc(c, b).wait()

                @pl.when(c >= 2 * NW)
                def _():
                    write_desc(c - 2 * NW, b).wait()

                transpose_chunk(b)
                write_desc(c, b).start()

        return carry

    kmax = -(-nch // NW)
    lax.fori_loop(0, (kmax + 1) // 2, pair, 0)

    write_desc(0, 0).wait()
    write_desc(0, 1).wait()


def _gather_body(table_hbm, idx_hbm, out_hbm, idx_v, rows_v, gsem, osem):
    wid = lax.axis_index("s") * NC + lax.axis_index("c")
    n_per_w = idx_hbm.shape[0] // NW
    base = wid * n_per_w
    nchunks = n_per_w // CHUNK

    pltpu.sync_copy(idx_hbm.at[pl.ds(base, n_per_w)], idx_v)

    def gather_desc(chunk, b):
        return pltpu.make_async_copy(
            table_hbm.at[idx_v.at[pl.ds(chunk * CHUNK, CHUNK)]],
            rows_v.at[b],
            gsem.at[b],
        )

    def out_desc(chunk, b):
        return pltpu.make_async_copy(
            rows_v.at[b],
            out_hbm.at[pl.ds(base + chunk * CHUNK, CHUNK)],
            osem.at[b],
        )

    for b in range(NBUF):
        gather_desc(b, b).start()

    def group(g, carry):
        for b in range(NBUF):
            i = g * NBUF + b
            gather_desc(i, b).wait()
            od = out_desc(i, b)
            od.start()
            nxt = i + NBUF

            @pl.when(nxt < nchunks)
            def _():
                od.wait()
                gather_desc(nxt, b).start()

        return carry

    lax.fori_loop(0, nchunks // NBUF, group, 0)

    for b in range(NBUF):
        out_desc(nchunks - NBUF + b, b).wait()


def kernel(actions, table):
    B, T = actions.shape
    V = table.shape[0]
    vpad = -(-V // TCH) * TCH
    n = B * T
    n_per_w = n // NW
    flat = actions.reshape(n)
    mesh = plsc.VectorSubcoreMesh(core_axis_name="c", subcore_axis_name="s")

    transpose = functools.partial(
        pl.kernel,
        mesh=mesh,
        out_type=jax.ShapeDtypeStruct((vpad, PADH), jnp.float32),
        scratch_types=[
            pltpu.VMEM((2, HID, TCH), jnp.float32),
            pltpu.VMEM((2, TCH, PADH), jnp.float32),
            pltpu.SemaphoreType.DMA((2,)),
            pltpu.SemaphoreType.DMA((2,)),
        ],
        compiler_params=pltpu.CompilerParams(needs_layout_passes=False),
    )(_transpose_body)
    wide = transpose(table.T)

    gather = functools.partial(
        pl.kernel,
        mesh=mesh,
        out_type=jax.ShapeDtypeStruct((n, PADH), jnp.float32),
        scratch_types=[
            pltpu.VMEM((n_per_w,), jnp.int32),
            pltpu.VMEM((NBUF, CHUNK, PADH), jnp.float32),
            pltpu.SemaphoreType.DMA((NBUF,)),
            pltpu.SemaphoreType.DMA((NBUF,)),
        ],
    )(_gather_body)
    out = gather(wide, flat)
    return out[:, :HID].reshape(B, T, HID)


# speedup vs baseline: 1.9531x; 1.9531x over previous
import functools

import jax
import jax.numpy as jnp
from jax import lax
from jax.experimental import pallas as pl
from jax.experimental.pallas import tpu as pltpu
from jax.experimental.pallas import tpu_sc as plsc

HID = 64
PADH = 128
NC = 2
NS = 16
NW = NC * NS
CHUNK = 200
NBUF = 4


def _gather_body(table_hbm, idx_hbm, out_hbm, idx_v, rows_v, gsem, osem):
    wid = lax.axis_index("s") * NC + lax.axis_index("c")
    n_per_w = idx_hbm.shape[0] // NW
    base = wid * n_per_w
    nchunks = n_per_w // CHUNK

    pltpu.sync_copy(idx_hbm.at[pl.ds(base, n_per_w)], idx_v)

    def gather_desc(chunk, b):
        return pltpu.make_async_copy(
            table_hbm.at[idx_v.at[pl.ds(chunk * CHUNK, CHUNK)]],
            rows_v.at[b],
            gsem.at[b],
        )

    def out_desc(chunk, b):
        return pltpu.make_async_copy(
            rows_v.at[b],
            out_hbm.at[pl.ds(base + chunk * CHUNK, CHUNK)],
            osem.at[b],
        )

    for b in range(NBUF):
        gather_desc(b, b).start()

    def group(g, carry):
        for b in range(NBUF):
            i = g * NBUF + b
            gather_desc(i, b).wait()
            od = out_desc(i, b)
            od.start()
            nxt = i + NBUF

            @pl.when(nxt < nchunks)
            def _():
                od.wait()
                gather_desc(nxt, b).start()

        return carry

    lax.fori_loop(0, nchunks // NBUF, group, 0)

    for b in range(NBUF):
        out_desc(nchunks - NBUF + b, b).wait()


def kernel(actions, table):
    B, T = actions.shape
    n = B * T
    n_per_w = n // NW
    flat = actions.reshape(n)
    padded = jnp.pad(table, ((0, 0), (0, PADH - HID)))
    mesh = plsc.VectorSubcoreMesh(core_axis_name="c", subcore_axis_name="s")
    gather = functools.partial(
        pl.kernel,
        mesh=mesh,
        out_type=jax.ShapeDtypeStruct((n, PADH), jnp.float32),
        scratch_types=[
            pltpu.VMEM((n_per_w,), jnp.int32),
            pltpu.VMEM((NBUF, CHUNK, PADH), jnp.float32),
            pltpu.SemaphoreType.DMA((NBUF,)),
            pltpu.SemaphoreType.DMA((NBUF,)),
        ],
    )(_gather_body)
    out = gather(padded, flat)
    return out[:, :HID].reshape(B, T, HID)
